# baseline (speedup 1.0000x reference)
import functools

import jax
import jax.numpy as jnp
from jax import lax
from jax.experimental import pallas as pl
from jax.experimental.pallas import tpu as pltpu
from jax.experimental.pallas import tpu_sc as plsc

N, E, D = 10000, 320000, 128
NC, NS = 2, 16
NW = NC * NS
CH = 64
NCHUNK = 160
EPW = NCHUNK * CH
EPAD = NW * EPW
BHR = N + 16
RPW = 624
RTAIL = N - NS * RPW
WB = 208
KB = 4


def _linear(h, w, b):
    def body(h_ref, w_ref, b_ref, o_ref):
        o_ref[...] = lax.dot_general(
            h_ref[...], w_ref[...], (((1,), (1,)), ((), ())),
            preferred_element_type=jnp.float32) + b_ref[...]

    return pl.pallas_call(
        body,
        out_shape=jax.ShapeDtypeStruct((N, D), jnp.float32),
    )(h, w, b.reshape(1, D))


def _sc_aggregate(Bh, src, dst):
    mesh = plsc.VectorSubcoreMesh(core_axis_name="c", subcore_axis_name="s")

    @functools.partial(
        pl.kernel,
        out_type=jax.ShapeDtypeStruct((NC, N, D), jnp.float32),
        mesh=mesh,
        scratch_types=[
            pltpu.VMEM((KB * 2 * CH,), jnp.int32),
            pltpu.VMEM((KB * 2 * CH,), jnp.int32),
            pltpu.VMEM((CH,), jnp.int32),
            pltpu.VMEM((CH,), jnp.int32),
            pltpu.VMEM((CH,), jnp.int32),
            pltpu.VMEM((CH,), jnp.int32),
            pltpu.VMEM((CH, D), jnp.float32),
            pltpu.VMEM((CH, D), jnp.float32),
            pltpu.VMEM((WB, D), jnp.float32),
            pltpu.VMEM_SHARED((N, D), jnp.float32),
            pltpu.SemaphoreType.DMA,
            pltpu.SemaphoreType.DMA,
        ],
    )
    def k(bh_hbm, src_hbm, dst_hbm, out_hbm,
          sidx, didx, sbuf0, sbuf1, dbuf0, dbuf1, rows0, rows1,
          zbuf, acc, sem0, sem1):
        cid = lax.axis_index("c")
        sid = lax.axis_index("s")
        wid = cid * NS + sid

        @pl.loop(0, WB)
        def _(r):
            @pl.loop(0, D, step=16)
            def _(c):
                zbuf[r, pl.ds(c, 16)] = jnp.zeros((16,), jnp.float32)

        @pl.loop(0, RPW, step=WB)
        def _(r):
            pltpu.sync_copy(zbuf, acc.at[pl.ds(sid * RPW + r, WB)])

        @pl.when(sid == 0)
        def _():
            pltpu.sync_copy(zbuf.at[pl.ds(0, RTAIL)],
                            acc.at[pl.ds(NS * RPW, RTAIL)])

        plsc.subcore_barrier()

        def cp_row(src1d, j, dst1d):
            @pl.loop(0, CH, step=16)
            def _(c):
                dst1d[pl.ds(c, 16)] = src1d[pl.ds(j * CH + c, 16)]

        ebase = wid * EPW

        @pl.loop(0, NCHUNK, step=2 * KB)
        def _(j):
            pltpu.sync_copy(
                src_hbm.at[pl.ds(ebase + j * CH, KB * 2 * CH)], sidx)
            pltpu.sync_copy(
                dst_hbm.at[pl.ds(ebase + j * CH, KB * 2 * CH)], didx)

            @pl.loop(0, 2 * KB, step=2)
            def _(t):
                cp_row(sidx, t, sbuf0)
                c0 = pltpu.async_copy(bh_hbm.at[sbuf0], rows0, sem0)
                cp_row(sidx, t + 1, sbuf1)
                c1 = pltpu.async_copy(bh_hbm.at[sbuf1], rows1, sem1)
                cp_row(didx, t, dbuf0)
                cp_row(didx, t + 1, dbuf1)
                c0.wait()
                pltpu.sync_copy(rows0, acc.at[dbuf0], add=True)
                c1.wait()
                pltpu.sync_copy(rows1, acc.at[dbuf1], add=True)

        plsc.subcore_barrier()

        @pl.loop(0, RPW, step=WB)
        def _(r):
            pltpu.sync_copy(acc.at[pl.ds(sid * RPW + r, WB)],
                            out_hbm.at[cid, pl.ds(sid * RPW + r, WB)])

        @pl.when(sid == 0)
        def _():
            pltpu.sync_copy(acc.at[pl.ds(NS * RPW, RTAIL)],
                            out_hbm.at[cid, pl.ds(NS * RPW, RTAIL)])

    return k(Bh, src, dst)


def _epilogue(h, A_w, A_b, partials, gamma, beta):
    def body(h_ref, aw_ref, ab_ref, p_ref, g_ref, b_ref, o_ref):
        hv = h_ref[...]
        ah = lax.dot_general(
            hv, aw_ref[...], (((1,), (1,)), ((), ())),
            preferred_element_type=jnp.float32)
        hn = ah + ab_ref[...] + p_ref[0] + p_ref[1]
        mean = jnp.sum(hn, axis=0, keepdims=True) / N
        sq = jnp.sum(hn * hn, axis=0, keepdims=True) / N
        var = sq - mean * mean
        inv = lax.rsqrt(var + 1e-5) * g_ref[...]
        bn = (hn - mean) * inv + b_ref[...]
        o_ref[...] = hv + jnp.maximum(bn, 0.0)

    return pl.pallas_call(
        body,
        out_shape=jax.ShapeDtypeStruct((N, D), jnp.float32),
    )(h, A_w, A_b.reshape(1, D), partials, gamma.reshape(1, D),
      beta.reshape(1, D))


def kernel(h, edge_index, e, A_w, A_b, B_w, B_b, gamma, beta):
    Bh = _linear(h, B_w, B_b)
    Bh = jnp.concatenate([Bh, jnp.zeros((BHR - N, D), jnp.float32)])
    pad = EPAD - E
    src = jnp.concatenate([edge_index[0], jnp.full((pad,), N, jnp.int32)])
    dst = jnp.concatenate([edge_index[1], jnp.zeros((pad,), jnp.int32)])
    partials = _sc_aggregate(Bh, src, dst)
    hn = _epilogue(h, A_w, A_b, partials, gamma, beta)
    return (hn, e)


# speedup vs baseline: 1.1220x; 1.1220x over previous
import functools

import jax
import jax.numpy as jnp
from jax import lax
from jax.experimental import pallas as pl
from jax.experimental.pallas import tpu as pltpu
from jax.experimental.pallas import tpu_sc as plsc

N, E, D = 10000, 320000, 128
NC, NS = 2, 16
NW = NC * NS
CH = 64
NCHUNK = 160
EPW = NCHUNK * CH
EPAD = NW * EPW
BHR = N + 16
RPW = 624
RTAIL = N - NS * RPW
WB = 208


def _linear(h, w, b):
    def body(h_ref, w_ref, b_ref, o_ref):
        o_ref[...] = lax.dot_general(
            h_ref[...], w_ref[...], (((1,), (1,)), ((), ())),
            preferred_element_type=jnp.float32) + b_ref[...]

    return pl.pallas_call(
        body,
        out_shape=jax.ShapeDtypeStruct((N, D), jnp.float32),
    )(h, w, b.reshape(1, D))


def _sc_aggregate(Bh, src, dst):
    mesh = plsc.VectorSubcoreMesh(core_axis_name="c", subcore_axis_name="s")

    @functools.partial(
        pl.kernel,
        out_type=jax.ShapeDtypeStruct((NC, N, D), jnp.float32),
        mesh=mesh,
        scratch_types=[
            pltpu.VMEM((CH,), jnp.int32),
            pltpu.VMEM((CH,), jnp.int32),
            pltpu.VMEM((CH,), jnp.int32),
            pltpu.VMEM((CH,), jnp.int32),
            pltpu.VMEM((CH, D), jnp.float32),
            pltpu.VMEM((CH, D), jnp.float32),
            pltpu.VMEM((WB, D), jnp.float32),
            pltpu.VMEM_SHARED((N, D), jnp.float32),
            pltpu.SemaphoreType.DMA,
            pltpu.SemaphoreType.DMA,
        ],
    )
    def k(bh_hbm, src_hbm, dst_hbm, out_hbm,
          sidx0, sidx1, didx0, didx1, rows0, rows1,
          zbuf, acc, sem0, sem1):
        cid = lax.axis_index("c")
        sid = lax.axis_index("s")
        wid = cid * NS + sid

        @pl.loop(0, WB)
        def _(r):
            @pl.loop(0, D, step=16)
            def _(c):
                zbuf[r, pl.ds(c, 16)] = jnp.zeros((16,), jnp.float32)

        @pl.loop(0, RPW, step=WB)
        def _(r):
            pltpu.sync_copy(zbuf, acc.at[pl.ds(sid * RPW + r, WB)])

        @pl.when(sid == 0)
        def _():
            pltpu.sync_copy(zbuf.at[pl.ds(0, RTAIL)],
                            acc.at[pl.ds(NS * RPW, RTAIL)])

        plsc.subcore_barrier()

        ebase = wid * EPW

        pltpu.sync_copy(src_hbm.at[pl.ds(ebase, CH)], sidx0)
        pltpu.sync_copy(dst_hbm.at[pl.ds(ebase, CH)], didx0)
        pltpu.async_copy(bh_hbm.at[sidx0], rows0, sem0)
        pltpu.sync_copy(src_hbm.at[pl.ds(ebase + CH, CH)], sidx1)
        pltpu.sync_copy(dst_hbm.at[pl.ds(ebase + CH, CH)], didx1)
        pltpu.async_copy(bh_hbm.at[sidx1], rows1, sem1)

        @pl.loop(0, NCHUNK - 2, step=2)
        def _(j):
            off = ebase + (j + 2) * CH
            pltpu.make_async_copy(bh_hbm.at[sidx0], rows0, sem0).wait()
            pltpu.sync_copy(rows0, acc.at[didx0], add=True)
            pltpu.sync_copy(src_hbm.at[pl.ds(off, CH)], sidx0)
            pltpu.sync_copy(dst_hbm.at[pl.ds(off, CH)], didx0)
            pltpu.async_copy(bh_hbm.at[sidx0], rows0, sem0)
            pltpu.make_async_copy(bh_hbm.at[sidx1], rows1, sem1).wait()
            pltpu.sync_copy(rows1, acc.at[didx1], add=True)
            pltpu.sync_copy(src_hbm.at[pl.ds(off + CH, CH)], sidx1)
            pltpu.sync_copy(dst_hbm.at[pl.ds(off + CH, CH)], didx1)
            pltpu.async_copy(bh_hbm.at[sidx1], rows1, sem1)

        pltpu.make_async_copy(bh_hbm.at[sidx0], rows0, sem0).wait()
        pltpu.sync_copy(rows0, acc.at[didx0], add=True)
        pltpu.make_async_copy(bh_hbm.at[sidx1], rows1, sem1).wait()
        pltpu.sync_copy(rows1, acc.at[didx1], add=True)

        plsc.subcore_barrier()

        @pl.loop(0, RPW, step=WB)
        def _(r):
            pltpu.sync_copy(acc.at[pl.ds(sid * RPW + r, WB)],
                            out_hbm.at[cid, pl.ds(sid * RPW + r, WB)])

        @pl.when(sid == 0)
        def _():
            pltpu.sync_copy(acc.at[pl.ds(NS * RPW, RTAIL)],
                            out_hbm.at[cid, pl.ds(NS * RPW, RTAIL)])

    return k(Bh, src, dst)


def _epilogue(h, A_w, A_b, partials, gamma, beta):
    def body(h_ref, aw_ref, ab_ref, p_ref, g_ref, b_ref, o_ref):
        hv = h_ref[...]
        ah = lax.dot_general(
            hv, aw_ref[...], (((1,), (1,)), ((), ())),
            preferred_element_type=jnp.float32)
        hn = ah + ab_ref[...] + p_ref[0] + p_ref[1]
        mean = jnp.sum(hn, axis=0, keepdims=True) / N
        sq = jnp.sum(hn * hn, axis=0, keepdims=True) / N
        var = sq - mean * mean
        inv = lax.rsqrt(var + 1e-5) * g_ref[...]
        bn = (hn - mean) * inv + b_ref[...]
        o_ref[...] = hv + jnp.maximum(bn, 0.0)

    return pl.pallas_call(
        body,
        out_shape=jax.ShapeDtypeStruct((N, D), jnp.float32),
    )(h, A_w, A_b.reshape(1, D), partials, gamma.reshape(1, D),
      beta.reshape(1, D))


def kernel(h, edge_index, e, A_w, A_b, B_w, B_b, gamma, beta):
    Bh = _linear(h, B_w, B_b)
    Bh = jnp.concatenate([Bh, jnp.zeros((BHR - N, D), jnp.float32)])
    pad = EPAD - E
    src = jnp.concatenate([edge_index[0], jnp.full((pad,), N, jnp.int32)])
    dst = jnp.concatenate([edge_index[1], jnp.zeros((pad,), jnp.int32)])
    partials = _sc_aggregate(Bh, src, dst)
    hn = _epilogue(h, A_w, A_b, partials, gamma, beta)
    return (hn, e)


# speedup vs baseline: 1.9848x; 1.7690x over previous
import functools

import jax
import jax.numpy as jnp
from jax import lax
from jax.experimental import pallas as pl
from jax.experimental.pallas import tpu as pltpu
from jax.experimental.pallas import tpu_sc as plsc

N, E, D = 10000, 320000, 128
NC, NS = 2, 16
NW = NC * NS
CH = 88
NCHUNK = 114
EPW = NCHUNK * CH
EPAD = NW * EPW
BHR = N + 16
RPW = 624
RTAIL = N - NS * RPW
WB = 208


def _linear(h, w, b):
    def body(h_ref, w_ref, b_ref, o_ref):
        o_ref[...] = lax.dot_general(
            h_ref[...], w_ref[...], (((1,), (1,)), ((), ())),
            preferred_element_type=jnp.float32) + b_ref[...]

    return pl.pallas_call(
        body,
        out_shape=jax.ShapeDtypeStruct((N, D), jnp.float32),
    )(h, w, b.reshape(1, D))


def _sc_aggregate(Bh, src, dst):
    mesh = plsc.VectorSubcoreMesh(core_axis_name="c", subcore_axis_name="s")

    @functools.partial(
        pl.kernel,
        out_type=jax.ShapeDtypeStruct((NC, N, D), jnp.float32),
        mesh=mesh,
        scratch_types=[
            pltpu.VMEM((CH,), jnp.int32),
            pltpu.VMEM((CH,), jnp.int32),
            pltpu.VMEM((CH,), jnp.int32),
            pltpu.VMEM((CH,), jnp.int32),
            pltpu.VMEM((CH, D), jnp.float32),
            pltpu.VMEM((CH, D), jnp.float32),
            pltpu.VMEM((WB, D), jnp.float32),
            pltpu.VMEM_SHARED((N, D), jnp.float32),
            pltpu.SemaphoreType.DMA,
            pltpu.SemaphoreType.DMA,
            pltpu.SemaphoreType.DMA,
            pltpu.SemaphoreType.DMA,
            pltpu.SemaphoreType.DMA,
            pltpu.SemaphoreType.DMA,
        ],
    )
    def k(bh_hbm, src_hbm, dst_hbm, out_hbm,
          sidx0, sidx1, didx0, didx1, rows0, rows1,
          zbuf, acc, sem0, sem1, is0, is1, is2, is3):
        cid = lax.axis_index("c")
        sid = lax.axis_index("s")
        wid = cid * NS + sid

        @pl.loop(0, WB)
        def _(r):
            @pl.loop(0, D, step=16)
            def _(c):
                zbuf[r, pl.ds(c, 16)] = jnp.zeros((16,), jnp.float32)

        @pl.loop(0, RPW, step=WB)
        def _(r):
            pltpu.sync_copy(zbuf, acc.at[pl.ds(sid * RPW + r, WB)])

        @pl.when(sid == 0)
        def _():
            pltpu.sync_copy(zbuf.at[pl.ds(0, RTAIL)],
                            acc.at[pl.ds(NS * RPW, RTAIL)])

        plsc.subcore_barrier()

        ebase = wid * EPW

        @pl.loop(0, NCHUNK, step=2)
        def _(j):
            off = ebase + j * CH
            i0 = pltpu.async_copy(src_hbm.at[pl.ds(off, CH)], sidx0, is0)
            i1 = pltpu.async_copy(dst_hbm.at[pl.ds(off, CH)], didx0, is1)
            i2 = pltpu.async_copy(src_hbm.at[pl.ds(off + CH, CH)], sidx1, is2)
            i3 = pltpu.async_copy(dst_hbm.at[pl.ds(off + CH, CH)], didx1, is3)
            i0.wait()
            c0 = pltpu.async_copy(bh_hbm.at[sidx0], rows0, sem0)
            i2.wait()
            c1 = pltpu.async_copy(bh_hbm.at[sidx1], rows1, sem1)
            i1.wait()
            i3.wait()
            c0.wait()
            pltpu.sync_copy(rows0, acc.at[didx0], add=True)
            c1.wait()
            pltpu.sync_copy(rows1, acc.at[didx1], add=True)

        plsc.subcore_barrier()

        @pl.loop(0, RPW, step=WB)
        def _(r):
            pltpu.sync_copy(acc.at[pl.ds(sid * RPW + r, WB)],
                            out_hbm.at[cid, pl.ds(sid * RPW + r, WB)])

        @pl.when(sid == 0)
        def _():
            pltpu.sync_copy(acc.at[pl.ds(NS * RPW, RTAIL)],
                            out_hbm.at[cid, pl.ds(NS * RPW, RTAIL)])

    return k(Bh, src, dst)


def _epilogue(h, A_w, A_b, partials, gamma, beta):
    def body(h_ref, aw_ref, ab_ref, p_ref, g_ref, b_ref, o_ref):
        hv = h_ref[...]
        ah = lax.dot_general(
            hv, aw_ref[...], (((1,), (1,)), ((), ())),
            preferred_element_type=jnp.float32)
        hn = ah + ab_ref[...] + p_ref[0] + p_ref[1]
        mean = jnp.sum(hn, axis=0, keepdims=True) / N
        sq = jnp.sum(hn * hn, axis=0, keepdims=True) / N
        var = sq - mean * mean
        inv = lax.rsqrt(var + 1e-5) * g_ref[...]
        bn = (hn - mean) * inv + b_ref[...]
        o_ref[...] = hv + jnp.maximum(bn, 0.0)

    return pl.pallas_call(
        body,
        out_shape=jax.ShapeDtypeStruct((N, D), jnp.float32),
    )(h, A_w, A_b.reshape(1, D), partials, gamma.reshape(1, D),
      beta.reshape(1, D))


def kernel(h, edge_index, e, A_w, A_b, B_w, B_b, gamma, beta):
    Bh = _linear(h, B_w, B_b)
    Bh = jnp.concatenate([Bh, jnp.zeros((BHR - N, D), jnp.float32)])
    pad = EPAD - E
    src = jnp.concatenate([edge_index[0], jnp.full((pad,), N, jnp.int32)])
    dst = jnp.concatenate([edge_index[1], jnp.zeros((pad,), jnp.int32)])
    partials = _sc_aggregate(Bh, src, dst)
    hn = _epilogue(h, A_w, A_b, partials, gamma, beta)
    return (hn, e)
